# baseline (speedup 1.0000x reference)
import functools

import numpy as np
import ml_dtypes
import jax
import jax.numpy as jnp
from jax.experimental import pallas as pl
from jax.experimental.pallas import tpu as pltpu

NUMANGLE = 180
NUMRHO = 184
R_PAD = 192
OUT_H = 128
OUT_W = 128
P = OUT_H * OUT_W

P_TILE = 2048
A_BLK = 12
K = A_BLK * R_PAD
N_A = NUMANGLE // A_BLK
N_P = P // P_TILE

_BF16_ONE = np.uint16(0x3F80)


def _rho_index_table(H, W, numangle, numrho):
    irho = float(int(np.sqrt(H * H + W * W) + 1)) / float(numrho - 1)
    angles = np.arange(numangle).astype(np.float64) * (np.pi / numangle)
    cosi = np.cos(angles) / irho
    sini = np.sin(angles) / irho
    xs = (np.arange(W) - W // 2).astype(np.float64)
    ys = (np.arange(H) - H // 2).astype(np.float64)
    r = np.round(
        cosi[:, None, None] * xs[None, None, :] + sini[:, None, None] * ys[None, :, None]
    ).astype(np.int32) + numrho // 2
    invalid = (r < 0) | (r >= numrho)
    r[invalid] = numrho
    return r.reshape(numangle, H * W)


_TABLE_CACHE = {}


def _onehot_table():
    if "t" not in _TABLE_CACHE:
        r = _rho_index_table(OUT_H, OUT_W, NUMANGLE, NUMRHO)
        ks = np.arange(R_PAD, dtype=np.int32)
        out = np.zeros((N_A, K, P), np.uint16)
        for t in range(N_A):
            for j in range(A_BLK):
                m = r[t * A_BLK + j][None, :] == ks[:, None]
                blk = out[t, j * R_PAD : (j + 1) * R_PAD]
                blk[m] = _BF16_ONE
        _TABLE_CACHE["t"] = out.view(ml_dtypes.bfloat16)
    return _TABLE_CACHE["t"]


def _idht_block(oh0_ref, oh1_ref, oh2_ref, acc_ref, out_ref):
    t = pl.program_id(1)
    kc = K // 3

    @pl.when(t == 0)
    def _zero():
        out_ref[...] = jnp.zeros_like(out_ref)

    for i, oh_ref in enumerate((oh0_ref, oh1_ref, oh2_ref)):
        out_ref[...] += jnp.dot(
            acc_ref[0, :, i * kc : (i + 1) * kc],
            oh_ref[0],
            preferred_element_type=jnp.float32,
        )


@functools.partial(jax.jit, static_argnames=("interpret",))
def kernel(accumulator, interpret=False):
    n, c, a_dim, r_dim = accumulator.shape
    nc = n * c
    oh = jnp.asarray(_onehot_table())
    acc_p = jnp.pad(
        accumulator.reshape(nc, a_dim, r_dim), ((0, 0), (0, 0), (0, R_PAD - r_dim))
    )
    acc_g = (
        acc_p.reshape(nc, N_A, K).transpose(1, 0, 2).astype(jnp.bfloat16)
    )

    kc = K // 3
    ohs = [oh[:, i * kc : (i + 1) * kc, :] for i in range(3)]
    out = pl.pallas_call(
        _idht_block,
        grid=(N_P, N_A),
        in_specs=[
            pl.BlockSpec((1, kc, P_TILE), lambda p, t: (t, 0, p)),
            pl.BlockSpec((1, kc, P_TILE), lambda p, t: (t, 0, p)),
            pl.BlockSpec((1, kc, P_TILE), lambda p, t: (t, 0, p)),
            pl.BlockSpec((1, nc, K), lambda p, t: (t, 0, 0)),
        ],
        out_specs=pl.BlockSpec((nc, P_TILE), lambda p, t: (0, p)),
        out_shape=jax.ShapeDtypeStruct((nc, P), jnp.float32),
        compiler_params=pltpu.CompilerParams(
            dimension_semantics=("parallel", "arbitrary"),
        ),
        interpret=interpret,
    )(*ohs, acc_g)

    return out.reshape(n, c, OUT_H, OUT_W)
